# Initial kernel scaffold; baseline (speedup 1.0000x reference)
import functools

import jax
import jax.numpy as jnp
from jax import lax
from jax.experimental import pallas as pl
from jax.experimental.pallas import tpu as pltpu
from jax.experimental.pallas import tpu_sc as plsc

_SCALE = 0.0625
_N = 512
_C = 256
_B = 2
_H = 32
_W = 32
_OUT_HW = 7
_NBINS = _OUT_HW * _OUT_HW
_OUTSZ = _C * _NBINS
_ROW = _W * _C
_NWORK = 32
_RPW = _N // _NWORK
_MAXNH = 6
_CCHUNK = _C // 16


def _ifloor(t):
    return t.astype(jnp.int32)


def _iceil(t):
    fi = t.astype(jnp.int32)
    return fi + (t > fi.astype(jnp.float32)).astype(jnp.int32)


def _round_half_even(t):
    fi = t.astype(jnp.int32)
    frac = t - fi.astype(jnp.float32)
    odd = jnp.bitwise_and(fi, 1)
    up = (frac > 0.5) | ((frac == 0.5) & (odd == 1))
    return fi + up.astype(jnp.int32)


def _body(feat_hbm, rois_hbm, out_hbm, rois_v, bounds_v, bounds_s,
          strip_v, out_v):
    wid = lax.axis_index("s") * 2 + lax.axis_index("c")
    iota = lax.iota(jnp.int32, 16)

    pltpu.sync_copy(rois_hbm.at[pl.ds(wid * (_RPW * 5), _RPW * 5)], rois_v)

    bidx = plsc.load_gather(rois_v, [iota * 5])
    x1 = plsc.load_gather(rois_v, [iota * 5 + 1])
    y1 = plsc.load_gather(rois_v, [iota * 5 + 2])
    x2 = plsc.load_gather(rois_v, [iota * 5 + 3])
    y2 = plsc.load_gather(rois_v, [iota * 5 + 4])

    rsw = _round_half_even(x1 * _SCALE)
    rsh = _round_half_even(y1 * _SCALE)
    rew = _round_half_even(x2 * _SCALE)
    reh = _round_half_even(y2 * _SCALE)
    roi_h = jnp.maximum(reh - rsh + 1, 1)
    roi_w = jnp.maximum(rew - rsw + 1, 1)
    bin_h = roi_h.astype(jnp.float32) / float(_OUT_HW)
    bin_w = roi_w.astype(jnp.float32) / float(_OUT_HW)

    bounds_v[0] = bidx.astype(jnp.int32) * _H
    for p in range(_OUT_HW):
        pf = float(p)
        hs = jnp.clip(_ifloor(pf * bin_h) + rsh, 0, _H)
        he = jnp.clip(_iceil((pf + 1.0) * bin_h) + rsh, 0, _H)
        ws = jnp.clip(_ifloor(pf * bin_w) + rsw, 0, _W)
        we = jnp.clip(_iceil((pf + 1.0) * bin_w) + rsw, 0, _W)
        bounds_v[1 + p] = hs
        bounds_v[8 + p] = he
        bounds_v[15 + p] = ws
        bounds_v[22 + p] = we

    pltpu.sync_copy(bounds_v, bounds_s)

    lane49 = iota * _NBINS
    minus_inf = jnp.full((16,), -jnp.inf, dtype=jnp.float32)

    def roi_body(r, carry):
        rowbase = bounds_s[0, r]

        for ph in range(_OUT_HW):
            hs = bounds_s[1 + ph, r]
            he = bounds_s[8 + ph, r]
            nh = he - hs

            def dma_row(dh, c):
                pltpu.sync_copy(feat_hbm.at[rowbase + hs + dh],
                                strip_v.at[dh])
                return c

            lax.fori_loop(0, nh, dma_row, 0)

            def pw_body(pw, c):
                ws = bounds_s[15 + pw, r]
                we = bounds_s[22 + pw, r]

                def w_body(w, accs):
                    def dh_body(dh, accs2):
                        base = w * _C
                        return tuple(
                            jnp.maximum(accs2[ci],
                                        strip_v[dh, pl.ds(base + ci * 16, 16)])
                            for ci in range(_CCHUNK))
                    return lax.fori_loop(0, nh, dh_body, accs)

                accs = lax.fori_loop(ws, we, w_body,
                                     (minus_inf,) * _CCHUNK)
                j = ph * _OUT_HW + pw
                for ci in range(_CCHUNK):
                    vals = jnp.where(accs[ci] == -jnp.inf,
                                     jnp.float32(0.0), accs[ci])
                    plsc.store_scatter(
                        out_v, [lane49 + (ci * (16 * _NBINS) + j)], vals)
                return c

            lax.fori_loop(0, _OUT_HW, pw_body, 0)

        pltpu.sync_copy(out_v, out_hbm.at[wid * _RPW + r])
        return carry

    lax.fori_loop(0, _RPW, roi_body, 0)


@jax.jit
def kernel(features, rois):
    feat = jnp.transpose(features, (0, 2, 3, 1)).reshape(_B * _H, _ROW)
    rois_flat = rois.reshape(_N * 5)

    mesh = plsc.VectorSubcoreMesh(core_axis_name="c", subcore_axis_name="s")
    run = functools.partial(
        pl.kernel,
        out_type=jax.ShapeDtypeStruct((_N, _OUTSZ), jnp.float32),
        mesh=mesh,
        scratch_types=[
            pltpu.VMEM((_RPW * 5,), jnp.float32),
            pltpu.VMEM((29, 16), jnp.int32),
            pltpu.SMEM((29, 16), jnp.int32),
            pltpu.VMEM((_MAXNH, _ROW), jnp.float32),
            pltpu.VMEM((_OUTSZ,), jnp.float32),
        ],
    )(_body)
    out = run(feat, rois_flat)
    return out.reshape(_N, _C, _OUT_HW, _OUT_HW)


# speedup vs baseline: 21.6692x; 21.6692x over previous
import dataclasses
import functools

import jax
import jax.numpy as jnp
import numpy as np
from jax import lax
from jax.experimental import pallas as pl
from jax.experimental.pallas import tpu as pltpu
from jax.experimental.pallas import tpu_sc as plsc

_SCALE = 0.0625
_N = 512
_C = 256
_B = 2
_H = 32
_W = 32
_OUT_HW = 7
_NBINS = _OUT_HW * _OUT_HW
_OUTSZ = _C * _NBINS
_ROW = _W * _C
_NWORK = 32
_RPW = _N // _NWORK
_MAXNH = 6
_CCHUNK = _C // 16


def _ifloor(t):
    return t.astype(jnp.int32)


def _iceil(t):
    fi = t.astype(jnp.int32)
    return fi + (t > fi.astype(jnp.float32)).astype(jnp.int32)


def _round_half_even(t):
    fi = t.astype(jnp.int32)
    frac = t - fi.astype(jnp.float32)
    odd = jnp.bitwise_and(fi, 1)
    up = (frac > 0.5) | ((frac == 0.5) & (odd == 1))
    return fi + up.astype(jnp.int32)


def _body(feat_hbm, rois_hbm, out_hbm, rois_v, bounds_v,
          strip_v, out_v):
    wid = lax.axis_index("s") * 2 + lax.axis_index("c")
    iota = lax.iota(jnp.int32, 16)

    pltpu.sync_copy(rois_hbm.at[pl.ds(wid * (_RPW * 5), _RPW * 5)], rois_v)

    bidx = plsc.load_gather(rois_v, [iota * 5])
    x1 = plsc.load_gather(rois_v, [iota * 5 + 1])
    y1 = plsc.load_gather(rois_v, [iota * 5 + 2])
    x2 = plsc.load_gather(rois_v, [iota * 5 + 3])
    y2 = plsc.load_gather(rois_v, [iota * 5 + 4])

    rsw = _round_half_even(x1 * _SCALE)
    rsh = _round_half_even(y1 * _SCALE)
    rew = _round_half_even(x2 * _SCALE)
    reh = _round_half_even(y2 * _SCALE)
    roi_h = jnp.maximum(reh - rsh + 1, 1)
    roi_w = jnp.maximum(rew - rsw + 1, 1)
    rcp7 = jnp.float32(np.float32(1.0) / np.float32(7.0))
    bin_h = roi_h.astype(jnp.float32) * rcp7
    bin_w = roi_w.astype(jnp.float32) * rcp7

    bounds_v[pl.ds(0, 16)] = bidx.astype(jnp.int32) * _H
    for p in range(_OUT_HW):
        pf = float(p)
        hs = jnp.clip(_ifloor(pf * bin_h) + rsh, 0, _H)
        he = jnp.clip(_iceil((pf + 1.0) * bin_h) + rsh, 0, _H)
        ws = jnp.clip(_ifloor(pf * bin_w) + rsw, 0, _W)
        we = jnp.clip(_iceil((pf + 1.0) * bin_w) + rsw, 0, _W)
        bounds_v[pl.ds((1 + p) * 16, 16)] = hs
        bounds_v[pl.ds((8 + p) * 16, 16)] = he
        bounds_v[pl.ds((15 + p) * 16, 16)] = ws
        bounds_v[pl.ds((22 + p) * 16, 16)] = we


    lane49 = iota * _NBINS
    minus_inf = jnp.full((16,), -jnp.inf, dtype=jnp.float32)

    def _extract(row, r):
        vec = bounds_v[pl.ds(row * 16, 16)]
        masked = jnp.where(iota == r, vec, jnp.int32(-2147483648))
        return lax.reduce_max(masked, axes=(0,))

    def roi_body(r, carry):
        rowbase = _extract(0, r)

        for ph in range(_OUT_HW):
            hs = _extract(1 + ph, r)
            he = _extract(8 + ph, r)
            nh = he - hs

            def dma_row(dh, c):
                pltpu.sync_copy(feat_hbm.at[rowbase + hs + dh],
                                strip_v.at[dh])
                return c

            lax.fori_loop(0, nh, dma_row, 0)

            def pw_body(pw, c):
                ws = _extract(15 + pw, r)
                we = _extract(22 + pw, r)

                def w_body(w, accs):
                    def dh_body(dh, accs2):
                        base = w * _C
                        return tuple(
                            jnp.maximum(accs2[ci],
                                        strip_v[dh, pl.ds(base + ci * 16, 16)])
                            for ci in range(_CCHUNK))
                    return lax.fori_loop(0, nh, dh_body, accs)

                accs = lax.fori_loop(ws, we, w_body,
                                     (minus_inf,) * _CCHUNK)
                j = ph * _OUT_HW + pw
                for ci in range(_CCHUNK):
                    vals = jnp.where(accs[ci] == -jnp.inf,
                                     jnp.float32(0.0), accs[ci])
                    plsc.store_scatter(
                        out_v, [lane49 + (ci * (16 * _NBINS) + j)], vals)
                return c

            lax.fori_loop(0, _OUT_HW, pw_body, 0)

        pltpu.sync_copy(out_v, out_hbm.at[wid * _RPW + r])
        return carry

    lax.fori_loop(0, _RPW, roi_body, 0)


@jax.jit
def kernel(features, rois):
    feat = jnp.transpose(features, (0, 2, 3, 1)).reshape(_B * _H, _ROW)
    rois_flat = rois.reshape(_N * 5)

    mesh = plsc.VectorSubcoreMesh(core_axis_name="c", subcore_axis_name="s")
    cp = pltpu.CompilerParams()
    if "needs_layout_passes" in pltpu.CompilerParams.__dataclass_fields__:
        cp = dataclasses.replace(cp, needs_layout_passes=False)
    run = functools.partial(
        pl.kernel,
        compiler_params=cp,
        out_type=jax.ShapeDtypeStruct((_N, _OUTSZ), jnp.float32),
        mesh=mesh,
        scratch_types=[
            pltpu.VMEM((_RPW * 5,), jnp.float32),
            pltpu.VMEM((29 * 16,), jnp.int32),
            pltpu.VMEM((_MAXNH, _ROW), jnp.float32),
            pltpu.VMEM((_OUTSZ,), jnp.float32),
        ],
    )(_body)
    out = run(feat, rois_flat)
    return out.reshape(_N, _C, _OUT_HW, _OUT_HW)


# speedup vs baseline: 39.0536x; 1.8023x over previous
import dataclasses
import functools

import jax
import jax.numpy as jnp
import numpy as np
from jax import lax
from jax.experimental import pallas as pl
from jax.experimental.pallas import tpu as pltpu
from jax.experimental.pallas import tpu_sc as plsc

_SCALE = 0.0625
_N = 512
_C = 256
_B = 2
_H = 32
_W = 32
_OUT_HW = 7
_NBINS = _OUT_HW * _OUT_HW
_OUTSZ = _C * _NBINS
_ROW = _W * _C
_NWORK = 32
_RPW = _N // _NWORK
_MAXNH = 6
_CCHUNK = _C // 16


def _ifloor(t):
    return t.astype(jnp.int32)


def _iceil(t):
    fi = t.astype(jnp.int32)
    return fi + (t > fi.astype(jnp.float32)).astype(jnp.int32)


def _round_half_even(t):
    fi = t.astype(jnp.int32)
    frac = t - fi.astype(jnp.float32)
    odd = jnp.bitwise_and(fi, 1)
    up = (frac > 0.5) | ((frac == 0.5) & (odd == 1))
    return fi + up.astype(jnp.int32)


def _body(feat_hbm, rois_hbm, out_hbm, rois_v, bounds_v,
          strip_a, strip_b, out_v, sem_a, sem_b):
    wid = lax.axis_index("s") * 2 + lax.axis_index("c")
    iota = lax.iota(jnp.int32, 16)

    pltpu.sync_copy(rois_hbm.at[pl.ds(wid * (_RPW * 5), _RPW * 5)], rois_v)

    bidx = plsc.load_gather(rois_v, [iota * 5])
    x1 = plsc.load_gather(rois_v, [iota * 5 + 1])
    y1 = plsc.load_gather(rois_v, [iota * 5 + 2])
    x2 = plsc.load_gather(rois_v, [iota * 5 + 3])
    y2 = plsc.load_gather(rois_v, [iota * 5 + 4])

    rsw = _round_half_even(x1 * _SCALE)
    rsh = _round_half_even(y1 * _SCALE)
    rew = _round_half_even(x2 * _SCALE)
    reh = _round_half_even(y2 * _SCALE)
    roi_h = jnp.maximum(reh - rsh + 1, 1)
    roi_w = jnp.maximum(rew - rsw + 1, 1)
    rcp7 = jnp.float32(np.float32(1.0) / np.float32(7.0))
    bin_h = roi_h.astype(jnp.float32) * rcp7
    bin_w = roi_w.astype(jnp.float32) * rcp7

    bounds_v[pl.ds(0, 16)] = bidx.astype(jnp.int32) * _H
    for p in range(_OUT_HW):
        pf = float(p)
        hs = jnp.clip(_ifloor(pf * bin_h) + rsh, 0, _H)
        he = jnp.clip(_iceil((pf + 1.0) * bin_h) + rsh, 0, _H)
        ws = jnp.clip(_ifloor(pf * bin_w) + rsw, 0, _W)
        we = jnp.clip(_iceil((pf + 1.0) * bin_w) + rsw, 0, _W)
        bounds_v[pl.ds((1 + p) * 16, 16)] = hs
        bounds_v[pl.ds((8 + p) * 16, 16)] = he
        bounds_v[pl.ds((15 + p) * 16, 16)] = ws
        bounds_v[pl.ds((22 + p) * 16, 16)] = we


    lane49 = iota * _NBINS
    minus_inf = jnp.full((16,), -jnp.inf, dtype=jnp.float32)

    def _extract(row, r):
        vec = bounds_v[pl.ds(row * 16, 16)]
        masked = jnp.where(iota == r, vec, jnp.int32(-2147483648))
        return lax.reduce_max(masked, axes=(0,))

    bufs = (strip_a, strip_b)
    sems = (sem_a, sem_b)

    def roi_body(r, carry):
        rowbase = _extract(0, r)
        hs = [_extract(1 + p, r) for p in range(_OUT_HW)]
        he = [_extract(8 + p, r) for p in range(_OUT_HW)]
        nh = [he[p] - hs[p] for p in range(_OUT_HW)]

        def fire(p):
            buf, sem = bufs[p % 2], sems[p % 2]

            def f(dh, c):
                pltpu.make_async_copy(feat_hbm.at[rowbase + hs[p] + dh],
                                      buf.at[pl.ds(dh * _ROW, _ROW)], sem).start()
                return c

            lax.fori_loop(0, nh[p], f, 0)

        def drain(p):
            buf, sem = bufs[p % 2], sems[p % 2]

            def f(dh, c):
                pltpu.make_async_copy(feat_hbm.at[rowbase],
                                      buf.at[pl.ds(0, _ROW)], sem).wait()
                return c

            lax.fori_loop(0, nh[p], f, 0)

        fire(0)
        for ph in range(_OUT_HW):
            strip_v = bufs[ph % 2]
            if ph + 1 < _OUT_HW:
                fire(ph + 1)
            drain(ph)
            nh_ph = nh[ph]

            def pw_body(pw, c):
                ws = _extract(15 + pw, r)
                we = _extract(22 + pw, r)

                def w_body(w, accs):
                    def dh_body(dh, accs2):
                        base = dh * _ROW + w * _C
                        return tuple(
                            jnp.maximum(accs2[ci],
                                        strip_v[pl.ds(base + ci * 16, 16)])
                            for ci in range(_CCHUNK))
                    return lax.fori_loop(0, nh_ph, dh_body, accs)

                accs = lax.fori_loop(ws, we, w_body,
                                     (minus_inf,) * _CCHUNK)
                j = ph * _OUT_HW + pw
                for ci in range(_CCHUNK):
                    vals = jnp.where(accs[ci] == -jnp.inf,
                                     jnp.float32(0.0), accs[ci])
                    plsc.store_scatter(
                        out_v, [lane49 + (ci * (16 * _NBINS) + j)], vals)
                return c

            lax.fori_loop(0, _OUT_HW, pw_body, 0)

        pltpu.sync_copy(out_v, out_hbm.at[wid * _RPW + r])
        return carry

    lax.fori_loop(0, _RPW, roi_body, 0)


@jax.jit
def kernel(features, rois):
    feat = jnp.transpose(features, (0, 2, 3, 1)).reshape(_B * _H, _ROW)
    rois_flat = rois.reshape(_N * 5)

    mesh = plsc.VectorSubcoreMesh(core_axis_name="c", subcore_axis_name="s")
    cp = pltpu.CompilerParams()
    if "needs_layout_passes" in pltpu.CompilerParams.__dataclass_fields__:
        cp = dataclasses.replace(cp, needs_layout_passes=False)
    run = functools.partial(
        pl.kernel,
        compiler_params=cp,
        out_type=jax.ShapeDtypeStruct((_N, _OUTSZ), jnp.float32),
        mesh=mesh,
        scratch_types=[
            pltpu.VMEM((_RPW * 5,), jnp.float32),
            pltpu.VMEM((29 * 16,), jnp.int32),
            pltpu.VMEM((_MAXNH * _ROW,), jnp.float32),
            pltpu.VMEM((_MAXNH * _ROW,), jnp.float32),
            pltpu.VMEM((_OUTSZ,), jnp.float32),
            pltpu.SemaphoreType.DMA,
            pltpu.SemaphoreType.DMA,
        ],
    )(_body)
    out = run(feat, rois_flat)
    return out.reshape(_N, _C, _OUT_HW, _OUT_HW)
